# Initial kernel scaffold; baseline (speedup 1.0000x reference)
import jax
import jax.numpy as jnp
from jax.experimental import pallas as pl


def kernel(x, boundaries, out_int32, right, out):
    raise NotImplementedError("write your pallas kernel here")


# speedup vs baseline: 8.4143x; 8.4143x over previous
import functools

import jax
import jax.numpy as jnp
from jax import lax
from jax.experimental import pallas as pl
from jax.experimental.pallas import tpu as pltpu
from jax.experimental.pallas import tpu_sc as plsc

L = 16
K = 128
S1 = 65536
S2 = 512
S3 = 4
W4 = 4
N = 8388608


@functools.partial(
    pl.kernel,
    out_type=jax.ShapeDtypeStruct((L,), jnp.int32),
    mesh=plsc.VectorSubcoreMesh(core_axis_name="c", subcore_axis_name="s"),
    scratch_types=[
        pltpu.VMEM((K,), jnp.int32),
        pltpu.VMEM((K,), jnp.float32),
        pltpu.VMEM((L,), jnp.int32),
        pltpu.VMEM((L,), jnp.float32),
        pltpu.VMEM((L,), jnp.float32),
        pltpu.VMEM((L,), jnp.int32),
        pltpu.VMEM((L,), jnp.int32),
        pltpu.SemaphoreType.DMA,
    ],
)
def _search(xs_hbm, rs_hbm, bounds_hbm, out_hbm,
            idx_v, vals_v, fidx_v, fvals_v, xs_v, rs_v, out_v, sem):
    only_tile0 = jnp.logical_and(lax.axis_index("c") == 0,
                                 lax.axis_index("s") == 0)

    @pl.when(only_tile0)
    def _():
        pltpu.sync_copy(xs_hbm, xs_v)
        pltpu.sync_copy(rs_hbm, rs_v)
        xv = xs_v[...]
        rv = rs_v[...]
        iota = lax.iota(jnp.int32, L)
        ones = jnp.ones((L,), jnp.int32)
        zeros = jnp.zeros((L,), jnp.int32)

        base = jnp.int32(0)
        for stride in (S1, S2, S3):
            bb = jnp.full((L,), base, jnp.int32)
            for k in range(K // L):
                idx_v[pl.ds(k * L, L)] = bb + (k * L + iota) * stride
            pltpu.async_copy(bounds_hbm.at[idx_v], vals_v, sem).wait()
            acc = zeros
            for k in range(K // L):
                b = vals_v[pl.ds(k * L, L)]
                le = jnp.where(b <= xv, ones, zeros)
                lt = jnp.where(b < xv, ones, zeros)
                acc = acc + jnp.where(rv != 0, le, lt)
            c = acc[0]
            for j in range(1, L):
                c = c + acc[j]
            base = base + jnp.maximum(c - 1, 0) * stride

        bb = jnp.full((L,), base, jnp.int32)
        fidx_v[...] = jnp.minimum(bb + iota, jnp.full((L,), N - 1, jnp.int32))
        pltpu.async_copy(bounds_hbm.at[fidx_v], fvals_v, sem).wait()
        fb = fvals_v[...]
        fle = jnp.where(fb <= xv, ones, zeros)
        flt = jnp.where(fb < xv, ones, zeros)
        facc = jnp.where(rv != 0, fle, flt)
        idx = base
        for j in range(W4):
            idx = idx + facc[j]

        out_v[...] = jnp.full((L,), idx, jnp.int32)
        pltpu.sync_copy(out_v, out_hbm)


def kernel(x, boundaries, out_int32, right, out):
    xq = jnp.asarray(x, dtype=boundaries.dtype)
    xs = jnp.full((L,), xq, dtype=boundaries.dtype)
    rs = jnp.full((L,), jnp.asarray(right, jnp.int32))
    res = _search(xs, rs, boundaries)
    return res[0].astype(jnp.int32)


# speedup vs baseline: 8.9205x; 1.0602x over previous
import functools

import jax
import jax.numpy as jnp
from jax import lax
from jax.experimental import pallas as pl
from jax.experimental.pallas import tpu as pltpu
from jax.experimental.pallas import tpu_sc as plsc

L = 16
K = 128
S1 = 65536
S2 = 512
W3 = 512
N = 8388608


@functools.partial(
    pl.kernel,
    out_type=jax.ShapeDtypeStruct((L,), jnp.int32),
    mesh=plsc.VectorSubcoreMesh(core_axis_name="c", subcore_axis_name="s"),
    scratch_types=[
        pltpu.VMEM((K,), jnp.int32),
        pltpu.VMEM((K,), jnp.float32),
        pltpu.VMEM((W3,), jnp.float32),
        pltpu.VMEM((2 * L,), jnp.float32),
        pltpu.VMEM((L,), jnp.int32),
        pltpu.SemaphoreType.DMA,
        pltpu.SemaphoreType.DMA,
    ],
)
def _search(params_hbm, bounds_hbm, out_hbm,
            idx_v, vals_v, last_v, par_v, out_v, sem, sem2):
    only_tile0 = jnp.logical_and(lax.axis_index("c") == 0,
                                 lax.axis_index("s") == 0)

    @pl.when(only_tile0)
    def _():
        iota = lax.iota(jnp.int32, L)
        ones = jnp.ones((L,), jnp.int32)
        zeros = jnp.zeros((L,), jnp.int32)

        for k in range(K // L):
            idx_v[pl.ds(k * L, L)] = (k * L + iota) * S1
        l1 = pltpu.async_copy(bounds_hbm.at[idx_v], vals_v, sem)
        pc = pltpu.async_copy(params_hbm, par_v, sem2)
        pc.wait()
        xv = par_v[pl.ds(0, L)]
        rvf = par_v[pl.ds(L, L)]
        l1.wait()

        def count(ref, nvec):
            acc = zeros
            for k in range(nvec):
                b = ref[pl.ds(k * L, L)]
                le = jnp.where(b <= xv, ones, zeros)
                lt = jnp.where(b < xv, ones, zeros)
                acc = acc + jnp.where(rvf != 0.0, le, lt)
            c = acc[0]
            for j in range(1, L):
                c = c + acc[j]
            return c

        c1 = count(vals_v, K // L)
        base = jnp.maximum(c1 - 1, 0) * S1

        bb = jnp.full((L,), base, jnp.int32)
        for k in range(K // L):
            idx_v[pl.ds(k * L, L)] = bb + (k * L + iota) * S2
        pltpu.async_copy(bounds_hbm.at[idx_v], vals_v, sem).wait()
        c2 = count(vals_v, K // L)
        base = base + jnp.maximum(c2 - 1, 0) * S2

        pltpu.async_copy(bounds_hbm.at[pl.ds(base, W3)], last_v, sem).wait()
        c3 = count(last_v, W3 // L)
        idx = base + c3

        out_v[...] = jnp.full((L,), idx, jnp.int32)
        pltpu.sync_copy(out_v, out_hbm)


def kernel(x, boundaries, out_int32, right, out):
    xq = jnp.asarray(x, dtype=boundaries.dtype)
    xs = jnp.full((L,), xq, dtype=boundaries.dtype)
    rf = jnp.full((L,), jnp.where(jnp.asarray(right, jnp.int32) != 0, 1.0, 0.0),
                  dtype=jnp.float32)
    params = jnp.concatenate([xs, rf])
    res = _search(params, boundaries)
    return res[0].astype(jnp.int32)


# speedup vs baseline: 9.5311x; 1.0685x over previous
import functools

import jax
import jax.numpy as jnp
from jax import lax
from jax.experimental import pallas as pl
from jax.experimental.pallas import tpu as pltpu
from jax.experimental.pallas import tpu_sc as plsc

L = 16
K = 128
S1 = 65536
S2 = 512
W3 = 512
N = 8388608


@functools.partial(
    pl.kernel,
    out_type=jax.ShapeDtypeStruct((L,), jnp.int32),
    mesh=plsc.VectorSubcoreMesh(core_axis_name="c", subcore_axis_name="s",
                                num_cores=1),
    scratch_types=[
        pltpu.VMEM((K,), jnp.int32),
        pltpu.VMEM((K,), jnp.float32),
        pltpu.VMEM((W3,), jnp.float32),
        pltpu.VMEM((2 * L,), jnp.float32),
        pltpu.VMEM((L,), jnp.int32),
        pltpu.SemaphoreType.DMA,
        pltpu.SemaphoreType.DMA,
    ],
)
def _search(params_hbm, bounds_hbm, out_hbm,
            idx_v, vals_v, last_v, par_v, out_v, sem, sem2):
    only_tile0 = jnp.logical_and(lax.axis_index("c") == 0,
                                 lax.axis_index("s") == 0)

    @pl.when(only_tile0)
    def _():
        iota = lax.iota(jnp.int32, L)
        ones = jnp.ones((L,), jnp.int32)
        zeros = jnp.zeros((L,), jnp.int32)

        for k in range(K // L):
            idx_v[pl.ds(k * L, L)] = (k * L + iota) * S1
        l1 = pltpu.async_copy(bounds_hbm.at[idx_v], vals_v, sem)
        pc = pltpu.async_copy(params_hbm, par_v, sem2)
        pc.wait()
        xv = par_v[pl.ds(0, L)]
        rvf = par_v[pl.ds(L, L)]
        l1.wait()

        def count(ref, nvec):
            acc = zeros
            for k in range(nvec):
                b = ref[pl.ds(k * L, L)]
                le = jnp.where(b <= xv, ones, zeros)
                lt = jnp.where(b < xv, ones, zeros)
                acc = acc + jnp.where(rvf != 0.0, le, lt)
            c = acc[0]
            for j in range(1, L):
                c = c + acc[j]
            return c

        c1 = count(vals_v, K // L)
        base = jnp.maximum(c1 - 1, 0) * S1

        bb = jnp.full((L,), base, jnp.int32)
        for k in range(K // L):
            idx_v[pl.ds(k * L, L)] = bb + (k * L + iota) * S2
        pltpu.async_copy(bounds_hbm.at[idx_v], vals_v, sem).wait()
        c2 = count(vals_v, K // L)
        base = base + jnp.maximum(c2 - 1, 0) * S2

        pltpu.async_copy(bounds_hbm.at[pl.ds(base, W3)], last_v, sem).wait()
        c3 = count(last_v, W3 // L)
        idx = base + c3

        out_v[...] = jnp.full((L,), idx, jnp.int32)
        pltpu.sync_copy(out_v, out_hbm)


def kernel(x, boundaries, out_int32, right, out):
    xq = jnp.asarray(x, dtype=boundaries.dtype)
    xs = jnp.full((L,), xq, dtype=boundaries.dtype)
    rf = jnp.full((L,), jnp.where(jnp.asarray(right, jnp.int32) != 0, 1.0, 0.0),
                  dtype=jnp.float32)
    params = jnp.concatenate([xs, rf])
    res = _search(params, boundaries)
    return res[0].astype(jnp.int32)


# speedup vs baseline: 9.6167x; 1.0090x over previous
import functools

import jax
import jax.numpy as jnp
from jax import lax
from jax.experimental import pallas as pl
from jax.experimental.pallas import tpu as pltpu
from jax.experimental.pallas import tpu_sc as plsc

L = 16
K = 128
S1 = 65536
S2 = 512
W3 = 512
N = 8388608


@functools.partial(
    pl.kernel,
    out_type=jax.ShapeDtypeStruct((L,), jnp.int32),
    mesh=plsc.VectorSubcoreMesh(core_axis_name="c", subcore_axis_name="s",
                                num_cores=1),
    scratch_types=[
        pltpu.VMEM((K,), jnp.int32),
        pltpu.VMEM((K,), jnp.float32),
        pltpu.VMEM((W3,), jnp.float32),
        pltpu.VMEM((L,), jnp.float32),
        pltpu.VMEM((L,), jnp.int32),
        pltpu.SemaphoreType.DMA,
        pltpu.SemaphoreType.DMA,
    ],
)
def _search(params_hbm, bounds_hbm, out_hbm,
            idx_v, vals_v, last_v, par_v, out_v, sem, sem2):
    only_tile0 = jnp.logical_and(lax.axis_index("c") == 0,
                                 lax.axis_index("s") == 0)

    @pl.when(only_tile0)
    def _():
        iota = lax.iota(jnp.int32, L)
        ones = jnp.ones((L,), jnp.int32)
        zeros = jnp.zeros((L,), jnp.int32)

        for k in range(K // L):
            idx_v[pl.ds(k * L, L)] = (k * L + iota) * S1
        l1 = pltpu.async_copy(bounds_hbm.at[idx_v], vals_v, sem)
        pc = pltpu.async_copy(params_hbm, par_v, sem2)
        pc.wait()
        xv = par_v[...]
        l1.wait()

        def count(ref, nvec):
            acc = zeros
            for k in range(nvec):
                b = ref[pl.ds(k * L, L)]
                acc = acc + jnp.where(b < xv, ones, zeros)
            lanes = [acc[j] for j in range(L)]
            while len(lanes) > 1:
                lanes = [lanes[i] + lanes[i + 1]
                         for i in range(0, len(lanes), 2)]
            return lanes[0]

        c1 = count(vals_v, K // L)
        base = jnp.maximum(c1 - 1, 0) * S1

        bb = jnp.full((L,), base, jnp.int32)
        for k in range(K // L):
            idx_v[pl.ds(k * L, L)] = bb + (k * L + iota) * S2
        pltpu.async_copy(bounds_hbm.at[idx_v], vals_v, sem).wait()
        c2 = count(vals_v, K // L)
        base = base + jnp.maximum(c2 - 1, 0) * S2

        pltpu.async_copy(bounds_hbm.at[pl.ds(base, W3)], last_v, sem).wait()
        c3 = count(last_v, W3 // L)
        idx = base + c3

        out_v[...] = jnp.full((L,), idx, jnp.int32)
        pltpu.sync_copy(out_v, out_hbm)


def kernel(x, boundaries, out_int32, right, out):
    xq = jnp.asarray(x, dtype=boundaries.dtype)
    xadj = jnp.where(jnp.asarray(right, jnp.int32) != 0,
                     jnp.nextafter(xq, jnp.inf), xq)
    params = jnp.full((L,), xadj, dtype=jnp.float32)
    res = _search(params, boundaries)
    return res[0].astype(jnp.int32)
